# baseline (speedup 1.0000x reference)
import functools

import jax
import jax.numpy as jnp
from jax import lax
from jax.experimental import pallas as pl
from jax.experimental.pallas import tpu as pltpu
from jax.experimental.pallas import tpu_sc as plsc

B = 16
N = 20000
TOP = 100
PAD = 112
NBUCKET = 256
CHUNKS = N // 16
CAP = 4096
NEG = -3.0e38
BIG = 0x7FFFFFFF
THR = 0.5

_mesh = plsc.VectorSubcoreMesh(core_axis_name="c", subcore_axis_name="s")


@functools.partial(
    pl.kernel,
    out_type=[
        jax.ShapeDtypeStruct((B, PAD), jnp.int32),
        jax.ShapeDtypeStruct((B, PAD), jnp.float32),
    ],
    mesh=_mesh,
    compiler_params=pltpu.CompilerParams(needs_layout_passes=False),
    scratch_types=[
        pltpu.VMEM((N,), jnp.float32),
        pltpu.VMEM((NBUCKET * 16,), jnp.int32),
        pltpu.VMEM((NBUCKET,), jnp.int32),
        pltpu.VMEM((CAP,), jnp.float32),
        pltpu.VMEM((CAP,), jnp.int32),
        pltpu.VMEM((PAD,), jnp.int32),
        pltpu.VMEM((PAD,), jnp.float32),
    ],
)
def _topk_sc(cls_hbm, idx_hbm, val_hbm,
             cls_v, hist, totals, cand_val, cand_idx, top_idx, top_val):
    wid = lax.axis_index("s") * 2 + lax.axis_index("c")

    @pl.when(wid < B)
    def _body():
        b = wid
        lane = jnp.arange(16, dtype=jnp.int32)
        ones_i = jnp.ones((16,), jnp.int32)
        zeros_f = jnp.zeros((16,), jnp.float32)
        neg_f = jnp.full((16,), NEG, jnp.float32)

        pltpu.sync_copy(cls_hbm.at[b], cls_v)

        def _zh(k, _):
            hist[pl.ds(k * 16, 16)] = jnp.zeros((16,), jnp.int32)
            return 0
        lax.fori_loop(0, NBUCKET, _zh, 0)

        def _bucket(v):
            bk = (v * jnp.float32(NBUCKET)).astype(jnp.int32)
            return jnp.clip(bk, 0, NBUCKET - 1)

        def _hb(c, _):
            base = c * 80
            for u in range(5):
                v = cls_v[pl.ds(base + u * 16, 16)]
                bk = _bucket(v)
                plsc.addupdate_scatter(hist, [bk * 16 + lane], ones_i)
            return 0
        lax.fori_loop(0, CHUNKS // 5, _hb, 0)

        def _tt(kc, _):
            acc = jnp.zeros((16,), jnp.int32)
            jbase = (kc * 16 + lane) * 16
            for l in range(16):
                acc = acc + plsc.load_gather(hist, [jbase + l])
            totals[pl.ds(kc * 16, 16)] = acc
            return 0
        lax.fori_loop(0, NBUCKET // 16, _tt, 0)

        def _fb(k, carry):
            cum, bsel = carry
            kk = NBUCKET - 1 - k
            t = plsc.load_gather(totals, [jnp.full((16,), kk, jnp.int32)])
            cum2 = cum + jnp.max(t)
            bsel2 = jnp.where((cum < TOP) & (cum2 >= TOP), kk, bsel)
            return (cum2, bsel2)
        _, bsel = lax.fori_loop(0, NBUCKET, _fb,
                                (jnp.int32(0), jnp.int32(0)))

        def _cp(c, off):
            v = cls_v[pl.ds(c * 16, 16)]
            m = _bucket(v) >= bsel

            def _append(o):
                cs = plsc.cumsum(m.astype(jnp.int32))
                pos = jnp.minimum(o + cs - 1, CAP - 17)
                plsc.store_scatter(cand_val, [pos], v, mask=m)
                plsc.store_scatter(cand_idx, [pos], c * 16 + lane, mask=m)
                return jnp.minimum(o + cs[15], CAP - 17)

            return lax.cond(jnp.any(m), _append, lambda o: o, off)
        m_end = lax.fori_loop(0, CHUNKS, _cp, jnp.int32(0))

        plsc.store_scatter(cand_val, [m_end + lane], neg_f)
        nch = (m_end + 16) // 16

        for c in range(PAD // 16):
            top_idx[pl.ds(c * 16, 16)] = jnp.zeros((16,), jnp.int32)
            top_val[pl.ds(c * 16, 16)] = zeros_f

        def _sel(k, _):
            def _mx(c, bvbp):
                bv, bp = bvbp
                v = cand_val[pl.ds(c * 16, 16)]
                p = c * 16 + lane
                gt = v > bv
                return (jnp.where(gt, v, bv), jnp.where(gt, p, bp))
            bv, bp = lax.fori_loop(0, nch, _mx,
                                   (neg_f, jnp.zeros((16,), jnp.int32)))
            maxv = jnp.max(bv)
            pm = jnp.where(bv == maxv, bp, BIG)
            minpos = jnp.full((16,), jnp.min(pm), jnp.int32)
            gi = plsc.load_gather(cand_idx, [minpos])
            l0 = lane == 0
            kk = jnp.full((16,), k, jnp.int32)
            plsc.store_scatter(top_idx, [kk], gi, mask=l0)
            plsc.store_scatter(top_val, [kk],
                               jnp.full((16,), maxv, jnp.float32), mask=l0)
            plsc.store_scatter(cand_val, [minpos], neg_f, mask=l0)
            return 0
        lax.fori_loop(0, TOP, _sel, 0)

        pltpu.sync_copy(top_idx, idx_hbm.at[b])
        pltpu.sync_copy(top_val, val_hbm.at[b])


def _gnms_body(idx_sref, r_ref, v_ref, oreg_ref, ocls_ref, rows_s, soa_s,
               alive_s):
    b = pl.program_id(0)

    soa_s[...] = jnp.zeros((8, 128), jnp.float32)
    for j in range(PAD):
        ij = idx_sref[b * PAD + j]
        row = r_ref[0, pl.ds(ij, 1), :]
        rows_s[pl.ds(j, 1), pl.ds(0, 4)] = row
        for d in range(4):
            soa_s[pl.ds(d, 1), pl.ds(j, 1)] = row[:, d:d + 1]

    lanes = lax.broadcasted_iota(jnp.int32, (1, 128), 1)
    x1 = soa_s[pl.ds(0, 1), :]
    y1 = soa_s[pl.ds(1, 1), :]
    x2 = soa_s[pl.ds(2, 1), :]
    y2 = soa_s[pl.ds(3, 1), :]
    area = (x2 - x1) * (y2 - y1)
    alive0 = jnp.where(lanes < TOP, 1.0, 0.0).astype(jnp.float32)

    def _nms(i, alive):
        onehot = lanes == i

        def pick(v):
            return jnp.max(jnp.where(onehot, v, NEG))

        xi, yi, Xi, Yi, ai = pick(x1), pick(y1), pick(x2), pick(y2), pick(area)
        live_i = jnp.max(jnp.where(onehot, alive, 0.0)) > 0.5
        w = jnp.maximum(jnp.minimum(Xi, x2) - jnp.maximum(xi, x1), 0.0)
        h = jnp.maximum(jnp.minimum(Yi, y2) - jnp.maximum(yi, y1), 0.0)
        inter = w * h
        un = jnp.maximum(ai + area - inter, 1e-8)
        keep = (inter < THR * un) | onehot
        return jnp.where(live_i, jnp.where(keep, alive, 0.0), alive)

    alive = lax.fori_loop(0, TOP, _nms, alive0)
    alive_s[...] = alive

    ocls_ref[pl.ds(b, 1), :] = v_ref[pl.ds(b, 1), :] * alive[:, :PAD]
    for j in range(TOP):
        aj = alive_s[0, pl.ds(j, 1)]
        oreg_ref[0, j, pl.ds(0, 4)] = rows_s[j, pl.ds(0, 4)] * aj


_gnms_tc = pl.pallas_call(
    _gnms_body,
    grid_spec=pltpu.PrefetchScalarGridSpec(
        num_scalar_prefetch=1,
        grid=(B,),
        in_specs=[
            pl.BlockSpec((1, N, 4), lambda b, *_: (b, 0, 0)),
            pl.BlockSpec((B, PAD), lambda b, *_: (0, 0)),
        ],
        out_specs=[
            pl.BlockSpec((1, TOP, 4), lambda b, *_: (b, 0, 0)),
            pl.BlockSpec((B, PAD), lambda b, *_: (0, 0)),
        ],
        scratch_shapes=[
            pltpu.VMEM((PAD, 128), jnp.float32),
            pltpu.VMEM((8, 128), jnp.float32),
            pltpu.VMEM((1, 128), jnp.float32),
        ],
    ),
    out_shape=[
        jax.ShapeDtypeStruct((B, TOP, 4), jnp.float32),
        jax.ShapeDtypeStruct((B, PAD), jnp.float32),
    ],
)


def kernel(reg, cls):
    top_idx, top_val = _topk_sc(cls)
    out_reg, out_cls_pad = _gnms_tc(top_idx.reshape(-1), reg, top_val)
    return out_reg, out_cls_pad[:, :TOP]
